# baseline (speedup 1.0000x reference)
import functools

import jax
import jax.numpy as jnp
from jax import lax
from jax.experimental import pallas as pl
from jax.experimental.pallas import tpu as pltpu
from jax.experimental.pallas import tpu_sc as plsc

BN_EPS_ = 1e-5
CHUNK = 128
N_SUBCORES = 16
N_CORES = 2
BNROWS = 400


@functools.cache
def _make_edge_sum(e_pad: int, n_pad: int, with_deg: bool):
    per_tile = e_pad // (N_SUBCORES * CHUNK)
    rows_per_tile = n_pad // N_SUBCORES
    wb_chunks = rows_per_tile // CHUNK
    assert per_tile % 2 == 0
    mesh = plsc.VectorSubcoreMesh(core_axis_name="c", subcore_axis_name="s",
                                  num_cores=N_CORES, num_subcores=N_SUBCORES)

    half = per_tile // 2
    assert half % 2 == 0 and half % 8 == 0
    assert n_pad // CHUNK <= CHUNK

    out_type = [jax.ShapeDtypeStruct((N_CORES, n_pad, CHUNK), jnp.float32)]
    scratch = [
        pltpu.VMEM((CHUNK, CHUNK), jnp.float32),
        pltpu.VMEM((CHUNK, CHUNK), jnp.float32),
        pltpu.VMEM((half, CHUNK), jnp.int32),
        pltpu.VMEM((half, CHUNK), jnp.int32),
        pltpu.VMEM_SHARED((n_pad, CHUNK), jnp.float32),
        pltpu.SemaphoreType.DMA,
        pltpu.SemaphoreType.DMA,
        pltpu.SemaphoreType.DMA,
        pltpu.SemaphoreType.DMA,
    ]
    if with_deg:
        out_type.append(jax.ShapeDtypeStruct((N_CORES * n_pad,), jnp.float32))
        scratch += [
            pltpu.VMEM((CHUNK,), jnp.float32),
            pltpu.VMEM((rows_per_tile,), jnp.float32),
            pltpu.VMEM_SHARED((n_pad,), jnp.float32),
        ]

    @functools.partial(pl.kernel, out_type=out_type, mesh=mesh,
                       scratch_types=scratch)
    def edge_sum(h2_hbm, gidx_hbm, row_hbm, out_hbm, *rest):
        if with_deg:
            (deg_hbm, buf0, buf1, gidx_v, rows_v, acc, sem0, sem1, sem2, sem3,
             ones_v, degbuf_v, dega) = rest
        else:
            buf0, buf1, gidx_v, rows_v, acc, sem0, sem1, sem2, sem3 = rest

        HC = CHUNK // 2

        def start_gather(jj, buf, sa, sb):
            pltpu.async_copy(h2_hbm.at[gidx_v.at[jj, pl.ds(0, HC)]],
                             buf.at[pl.ds(0, HC)], sa)
            pltpu.async_copy(h2_hbm.at[gidx_v.at[jj, pl.ds(HC, HC)]],
                             buf.at[pl.ds(HC, HC)], sb)

        def wait_gather(buf, sa, sb):
            pltpu.make_async_copy(h2_hbm.at[pl.ds(0, HC)],
                                  buf.at[pl.ds(0, HC)], sa).wait()
            pltpu.make_async_copy(h2_hbm.at[pl.ds(0, HC)],
                                  buf.at[pl.ds(HC, HC)], sb).wait()
        c = lax.axis_index("c")
        s = lax.axis_index("s")
        base = s * per_tile

        @pl.loop(0, CHUNK)
        def _(r):
            for q in range(CHUNK // 16):
                buf0[r, pl.ds(q * 16, 16)] = jnp.zeros((16,), jnp.float32)

        if with_deg:
            for q in range(CHUNK // 16):
                ones_v[pl.ds(q * 16, 16)] = jnp.full((16,), 1.0, jnp.float32)

            @pl.loop(0, rows_per_tile // 16)
            def _(q):
                degbuf_v[pl.ds(q * 16, 16)] = jnp.zeros((16,), jnp.float32)

        for k in range(wb_chunks):
            pltpu.sync_copy(buf0, acc.at[pl.ds(s * rows_per_tile + k * CHUNK, CHUNK)])
        if with_deg:
            pltpu.sync_copy(degbuf_v, dega.at[pl.ds(s * rows_per_tile, rows_per_tile)])
        plsc.subcore_barrier()

        for ph in range(2):
            pltpu.sync_copy(gidx_hbm.at[c, pl.ds(base + ph * half, half)], gidx_v)
            pltpu.sync_copy(row_hbm.at[pl.ds(base + ph * half, half)], rows_v)

            start_gather(0, buf0, sem0, sem1)
            start_gather(1, buf1, sem2, sem3)

            @pl.loop(0, half - 2, step=2)
            def _(j):
                wait_gather(buf0, sem0, sem1)
                pltpu.sync_copy(buf0, acc.at[rows_v.at[j]], add=True)
                start_gather(j + 2, buf0, sem0, sem1)
                if with_deg:
                    @pl.when(c == 0)
                    def _():
                        pltpu.sync_copy(ones_v, dega.at[rows_v.at[j]], add=True)
                wait_gather(buf1, sem2, sem3)
                pltpu.sync_copy(buf1, acc.at[rows_v.at[j + 1]], add=True)
                start_gather(j + 3, buf1, sem2, sem3)
                if with_deg:
                    @pl.when(c == 1)
                    def _():
                        pltpu.sync_copy(ones_v, dega.at[rows_v.at[j + 1]], add=True)

            wait_gather(buf0, sem0, sem1)
            pltpu.sync_copy(buf0, acc.at[rows_v.at[half - 2]], add=True)
            wait_gather(buf1, sem2, sem3)
            pltpu.sync_copy(buf1, acc.at[rows_v.at[half - 1]], add=True)
            if with_deg:
                @pl.when(c == 0)
                def _():
                    pltpu.sync_copy(ones_v, dega.at[rows_v.at[half - 2]], add=True)

                @pl.when(c == 1)
                def _():
                    pltpu.sync_copy(ones_v, dega.at[rows_v.at[half - 1]], add=True)

        plsc.subcore_barrier()

        off = s * rows_per_tile
        pltpu.sync_copy(acc.at[pl.ds(off, rows_per_tile)],
                        out_hbm.at[c, pl.ds(off, rows_per_tile)])
        if with_deg:
            pltpu.sync_copy(dega.at[pl.ds(off, rows_per_tile)],
                            deg_hbm.at[pl.ds(c * n_pad + off, rows_per_tile)])

    return edge_sum


def _k1_body(x_ref, w_ref, ego_ref, xn_ref):
    r = jnp.dot(x_ref[...], w_ref[...], preferred_element_type=jnp.float32)
    ego_ref[...] = r[:, :256]
    xn_ref[...] = r[:, 256:]


def _k2_body(ego_ref, xn_ref, s0_ref, s1_ref, d0_ref, d1_ref, w1_ref, w2_ref,
             sc1_ref, sh1_ref, x1_ref, y2_ref):
    invd = 1.0 / (d0_ref[...] + d1_ref[...] + 1.0)
    xn = xn_ref[...]
    agg = (jnp.concatenate([s0_ref[...], s1_ref[...]], axis=1) + xn) * invd
    h1 = ego_ref[...] + agg
    t = jnp.maximum(jnp.dot(h1, w1_ref[...], preferred_element_type=jnp.float32), 0.0)
    x1 = t * sc1_ref[...] + sh1_ref[...]
    x1_ref[...] = x1
    y2_ref[...] = jnp.dot(x1, w2_ref[...], preferred_element_type=jnp.float32)


def _k3_body(x1_ref, y2_ref, s0_ref, s1_ref, d0_ref, d1_ref, wt_ref, wb_ref,
             b_ref, sc2_ref, sh2_ref, out_ref, *, n_classes):
    invd = 1.0 / (d0_ref[...] + d1_ref[...] + 1.0)
    y2 = y2_ref[...]
    agg = (jnp.concatenate([s0_ref[...], s1_ref[...]], axis=1) + y2) * invd
    x2 = jnp.maximum(agg, 0.0) * sc2_ref[...] + sh2_ref[...]
    logits = (jnp.dot(x1_ref[...], wt_ref[...], preferred_element_type=jnp.float32)
              + jnp.dot(x2, wb_ref[...], preferred_element_type=jnp.float32)
              + b_ref[...])
    mask = lax.broadcasted_iota(jnp.int32, logits.shape, 1) < n_classes
    neg = jnp.where(mask, logits, -1e30)
    m = jnp.max(neg, axis=1, keepdims=True)
    e = jnp.where(mask, jnp.exp(neg - m), 0.0)
    lse = jnp.log(jnp.sum(e, axis=1, keepdims=True))
    out_ref[...] = neg - m - lse


def _row_spec(width):
    return pl.BlockSpec((BNROWS, width), lambda i: (i, 0))


def _full_spec(shape):
    return pl.BlockSpec(shape, lambda i: (0, 0))


def kernel(x, edge_index, W_ego, W_neigh, W_hop1, W_hop2,
           bn1_gamma, bn1_beta, bn1_mean, bn1_var,
           bn2_gamma, bn2_beta, bn2_mean, bn2_var,
           W_cls, b_cls):
    N, D = x.shape
    H = W_ego.shape[1]
    C = W_cls.shape[1]
    E = edge_index.shape[1]
    grid = (N // BNROWS,)

    row = edge_index[0].astype(jnp.int32)
    col = edge_index[1].astype(jnp.int32)
    e_unit = N_SUBCORES * CHUNK * 8
    e_pad = ((E + e_unit - 1) // e_unit) * e_unit
    n_unit = N_SUBCORES * CHUNK
    n_pad = ((N + n_unit - 1) // n_unit) * n_unit
    rowp = jnp.concatenate([row, jnp.full((e_pad - E,), N, jnp.int32)])
    colp = jnp.concatenate([col, jnp.zeros((e_pad - E,), jnp.int32)])
    row2 = rowp.reshape(e_pad // CHUNK, CHUNK)
    gidx2 = ((colp * 2)[None, :]
             + jnp.arange(N_CORES, dtype=jnp.int32)[:, None]
             ).reshape(N_CORES, e_pad // CHUNK, CHUNK)

    w_cat = jnp.concatenate([W_ego, W_neigh], axis=1)
    s1 = (bn1_gamma / jnp.sqrt(bn1_var + BN_EPS_)).reshape(1, H)
    t1 = (bn1_beta - bn1_mean * s1[0]).reshape(1, H)
    s2 = (bn2_gamma / jnp.sqrt(bn2_var + BN_EPS_)).reshape(1, H)
    t2 = (bn2_beta - bn2_mean * s2[0]).reshape(1, H)
    wt = jnp.pad(W_cls[:H], ((0, 0), (0, 128 - C)))
    wb = jnp.pad(W_cls[H:], ((0, 0), (0, 128 - C)))
    bpad = jnp.pad(b_cls, (0, 128 - C)).reshape(1, 128)

    edge_sum_deg = _make_edge_sum(e_pad, n_pad, True)
    edge_sum_nodeg = _make_edge_sum(e_pad, n_pad, False)

    x_ego, xn = pl.pallas_call(
        _k1_body,
        grid=grid,
        in_specs=[_row_spec(D), _full_spec((D, 2 * H))],
        out_specs=[_row_spec(H), _row_spec(H)],
        out_shape=[jax.ShapeDtypeStruct((N, H), jnp.float32)] * 2,
    )(x, w_cat)

    sum1, deg2 = edge_sum_deg(xn.reshape(2 * N, 128), gidx2, row2)
    deg0 = deg2[:n_pad].reshape(n_pad, 1)
    deg1 = deg2[n_pad:].reshape(n_pad, 1)

    x1, y2 = pl.pallas_call(
        _k2_body,
        grid=grid,
        in_specs=[_row_spec(H), _row_spec(H), _row_spec(128), _row_spec(128),
                  _row_spec(1), _row_spec(1), _full_spec((H, H)),
                  _full_spec((H, H)), _full_spec((1, H)), _full_spec((1, H))],
        out_specs=[_row_spec(H), _row_spec(H)],
        out_shape=[jax.ShapeDtypeStruct((N, H), jnp.float32)] * 2,
    )(x_ego, xn, sum1[0], sum1[1], deg0, deg1, W_hop1, W_hop2, s1, t1)

    sum2 = edge_sum_nodeg(y2.reshape(2 * N, 128), gidx2, row2)
    if isinstance(sum2, (list, tuple)):
        sum2 = sum2[0]

    out = pl.pallas_call(
        functools.partial(_k3_body, n_classes=C),
        grid=grid,
        in_specs=[_row_spec(H), _row_spec(H), _row_spec(128), _row_spec(128),
                  _row_spec(1), _row_spec(1), _full_spec((H, 128)),
                  _full_spec((H, 128)), _full_spec((1, 128)),
                  _full_spec((1, H)), _full_spec((1, H))],
        out_specs=_row_spec(128),
        out_shape=jax.ShapeDtypeStruct((N, 128), jnp.float32),
    )(x1, y2, sum2[0], sum2[1], deg0, deg1, wt, wb, bpad, s2, t2)

    return out[:, :C]


# speedup vs baseline: 1.0492x; 1.0492x over previous
import functools

import jax
import jax.numpy as jnp
from jax import lax
from jax.experimental import pallas as pl
from jax.experimental.pallas import tpu as pltpu
from jax.experimental.pallas import tpu_sc as plsc

BN_EPS_ = 1e-5
CHUNK = 128
N_SUBCORES = 16
N_CORES = 2
BNROWS = 2000


@functools.cache
def _make_edge_sum(e_pad: int, n_pad: int, with_deg: bool):
    per_tile = e_pad // (N_SUBCORES * CHUNK)
    rows_per_tile = n_pad // N_SUBCORES
    wb_chunks = rows_per_tile // CHUNK
    assert per_tile % 2 == 0
    mesh = plsc.VectorSubcoreMesh(core_axis_name="c", subcore_axis_name="s",
                                  num_cores=N_CORES, num_subcores=N_SUBCORES)

    half = per_tile // 2
    assert half % 2 == 0 and half % 8 == 0
    assert n_pad // CHUNK <= CHUNK

    out_type = [jax.ShapeDtypeStruct((N_CORES, n_pad, CHUNK), jnp.float32)]
    scratch = [
        pltpu.VMEM((CHUNK, CHUNK), jnp.float32),
        pltpu.VMEM((CHUNK, CHUNK), jnp.float32),
        pltpu.VMEM((half, CHUNK), jnp.int32),
        pltpu.VMEM((half, CHUNK), jnp.int32),
        pltpu.VMEM_SHARED((n_pad, CHUNK), jnp.float32),
        pltpu.SemaphoreType.DMA,
        pltpu.SemaphoreType.DMA,
    ]
    if with_deg:
        out_type.append(jax.ShapeDtypeStruct((N_CORES * n_pad,), jnp.float32))
        scratch += [
            pltpu.VMEM((CHUNK,), jnp.float32),
            pltpu.VMEM((rows_per_tile,), jnp.float32),
            pltpu.VMEM_SHARED((n_pad,), jnp.float32),
        ]

    @functools.partial(pl.kernel, out_type=out_type, mesh=mesh,
                       scratch_types=scratch)
    def edge_sum(h2_hbm, gidx_hbm, row_hbm, out_hbm, *rest):
        if with_deg:
            (deg_hbm, buf0, buf1, gidx_v, rows_v, acc, sem0, sem1,
             ones_v, degbuf_v, dega) = rest
        else:
            buf0, buf1, gidx_v, rows_v, acc, sem0, sem1 = rest
        c = lax.axis_index("c")
        s = lax.axis_index("s")
        base = s * per_tile

        @pl.loop(0, CHUNK)
        def _(r):
            for q in range(CHUNK // 16):
                buf0[r, pl.ds(q * 16, 16)] = jnp.zeros((16,), jnp.float32)

        if with_deg:
            for q in range(CHUNK // 16):
                ones_v[pl.ds(q * 16, 16)] = jnp.full((16,), 1.0, jnp.float32)

            @pl.loop(0, rows_per_tile // 16)
            def _(q):
                degbuf_v[pl.ds(q * 16, 16)] = jnp.zeros((16,), jnp.float32)

        for k in range(wb_chunks):
            pltpu.sync_copy(buf0, acc.at[pl.ds(s * rows_per_tile + k * CHUNK, CHUNK)])
        if with_deg:
            pltpu.sync_copy(degbuf_v, dega.at[pl.ds(s * rows_per_tile, rows_per_tile)])
        plsc.subcore_barrier()

        for ph in range(2):
            pltpu.sync_copy(gidx_hbm.at[c, pl.ds(base + ph * half, half)], gidx_v)
            pltpu.sync_copy(row_hbm.at[pl.ds(base + ph * half, half)], rows_v)

            pltpu.async_copy(h2_hbm.at[gidx_v.at[0]], buf0, sem0)
            pltpu.async_copy(h2_hbm.at[gidx_v.at[1]], buf1, sem1)

            @pl.loop(0, half - 2, step=2)
            def _(j):
                pltpu.make_async_copy(h2_hbm.at[pl.ds(0, CHUNK)], buf0, sem0).wait()
                pltpu.sync_copy(buf0, acc.at[rows_v.at[j]], add=True)
                pltpu.async_copy(h2_hbm.at[gidx_v.at[j + 2]], buf0, sem0)
                if with_deg:
                    @pl.when(c == 0)
                    def _():
                        pltpu.sync_copy(ones_v, dega.at[rows_v.at[j]], add=True)
                pltpu.make_async_copy(h2_hbm.at[pl.ds(0, CHUNK)], buf1, sem1).wait()
                pltpu.sync_copy(buf1, acc.at[rows_v.at[j + 1]], add=True)
                pltpu.async_copy(h2_hbm.at[gidx_v.at[j + 3]], buf1, sem1)
                if with_deg:
                    @pl.when(c == 1)
                    def _():
                        pltpu.sync_copy(ones_v, dega.at[rows_v.at[j + 1]], add=True)

            pltpu.make_async_copy(h2_hbm.at[pl.ds(0, CHUNK)], buf0, sem0).wait()
            pltpu.sync_copy(buf0, acc.at[rows_v.at[half - 2]], add=True)
            pltpu.make_async_copy(h2_hbm.at[pl.ds(0, CHUNK)], buf1, sem1).wait()
            pltpu.sync_copy(buf1, acc.at[rows_v.at[half - 1]], add=True)
            if with_deg:
                @pl.when(c == 0)
                def _():
                    pltpu.sync_copy(ones_v, dega.at[rows_v.at[half - 2]], add=True)

                @pl.when(c == 1)
                def _():
                    pltpu.sync_copy(ones_v, dega.at[rows_v.at[half - 1]], add=True)

        plsc.subcore_barrier()

        off = s * rows_per_tile
        pltpu.sync_copy(acc.at[pl.ds(off, rows_per_tile)],
                        out_hbm.at[c, pl.ds(off, rows_per_tile)])
        if with_deg:
            pltpu.sync_copy(dega.at[pl.ds(off, rows_per_tile)],
                            deg_hbm.at[pl.ds(c * n_pad + off, rows_per_tile)])

    return edge_sum


def _k1_body(x_ref, w_ref, ego_ref, xn_ref):
    r = jnp.dot(x_ref[...], w_ref[...], preferred_element_type=jnp.float32)
    ego_ref[...] = r[:, :256]
    xn_ref[...] = r[:, 256:]


def _k2_body(ego_ref, xn_ref, s0_ref, s1_ref, d0_ref, d1_ref, w1_ref, w2_ref,
             sc1_ref, sh1_ref, x1_ref, y2_ref):
    invd = 1.0 / (d0_ref[...] + d1_ref[...] + 1.0)
    xn = xn_ref[...]
    agg = (jnp.concatenate([s0_ref[...], s1_ref[...]], axis=1) + xn) * invd
    h1 = ego_ref[...] + agg
    t = jnp.maximum(jnp.dot(h1, w1_ref[...], preferred_element_type=jnp.float32), 0.0)
    x1 = t * sc1_ref[...] + sh1_ref[...]
    x1_ref[...] = x1
    y2_ref[...] = jnp.dot(x1, w2_ref[...], preferred_element_type=jnp.float32)


def _k3_body(x1_ref, y2_ref, s0_ref, s1_ref, d0_ref, d1_ref, wt_ref, wb_ref,
             b_ref, sc2_ref, sh2_ref, out_ref, *, n_classes):
    invd = 1.0 / (d0_ref[...] + d1_ref[...] + 1.0)
    y2 = y2_ref[...]
    agg = (jnp.concatenate([s0_ref[...], s1_ref[...]], axis=1) + y2) * invd
    x2 = jnp.maximum(agg, 0.0) * sc2_ref[...] + sh2_ref[...]
    logits = (jnp.dot(x1_ref[...], wt_ref[...], preferred_element_type=jnp.float32)
              + jnp.dot(x2, wb_ref[...], preferred_element_type=jnp.float32)
              + b_ref[...])
    mask = lax.broadcasted_iota(jnp.int32, logits.shape, 1) < n_classes
    neg = jnp.where(mask, logits, -1e30)
    m = jnp.max(neg, axis=1, keepdims=True)
    e = jnp.where(mask, jnp.exp(neg - m), 0.0)
    lse = jnp.log(jnp.sum(e, axis=1, keepdims=True))
    out_ref[...] = neg - m - lse


def _row_spec(width):
    return pl.BlockSpec((BNROWS, width), lambda i: (i, 0))


def _full_spec(shape):
    return pl.BlockSpec(shape, lambda i: (0, 0))


def kernel(x, edge_index, W_ego, W_neigh, W_hop1, W_hop2,
           bn1_gamma, bn1_beta, bn1_mean, bn1_var,
           bn2_gamma, bn2_beta, bn2_mean, bn2_var,
           W_cls, b_cls):
    N, D = x.shape
    H = W_ego.shape[1]
    C = W_cls.shape[1]
    E = edge_index.shape[1]
    grid = (N // BNROWS,)

    row = edge_index[0].astype(jnp.int32)
    col = edge_index[1].astype(jnp.int32)
    e_unit = N_SUBCORES * CHUNK * 8
    e_pad = ((E + e_unit - 1) // e_unit) * e_unit
    n_unit = N_SUBCORES * CHUNK
    n_pad = ((N + n_unit - 1) // n_unit) * n_unit
    rowp = jnp.concatenate([row, jnp.full((e_pad - E,), N, jnp.int32)])
    colp = jnp.concatenate([col, jnp.zeros((e_pad - E,), jnp.int32)])
    row2 = rowp.reshape(e_pad // CHUNK, CHUNK)
    gidx2 = ((colp * 2)[None, :]
             + jnp.arange(N_CORES, dtype=jnp.int32)[:, None]
             ).reshape(N_CORES, e_pad // CHUNK, CHUNK)

    w_cat = jnp.concatenate([W_ego, W_neigh], axis=1)
    s1 = (bn1_gamma / jnp.sqrt(bn1_var + BN_EPS_)).reshape(1, H)
    t1 = (bn1_beta - bn1_mean * s1[0]).reshape(1, H)
    s2 = (bn2_gamma / jnp.sqrt(bn2_var + BN_EPS_)).reshape(1, H)
    t2 = (bn2_beta - bn2_mean * s2[0]).reshape(1, H)
    wt = jnp.pad(W_cls[:H], ((0, 0), (0, 128 - C)))
    wb = jnp.pad(W_cls[H:], ((0, 0), (0, 128 - C)))
    bpad = jnp.pad(b_cls, (0, 128 - C)).reshape(1, 128)

    edge_sum_deg = _make_edge_sum(e_pad, n_pad, True)
    edge_sum_nodeg = _make_edge_sum(e_pad, n_pad, False)

    x_ego, xn = pl.pallas_call(
        _k1_body,
        grid=grid,
        in_specs=[_row_spec(D), _full_spec((D, 2 * H))],
        out_specs=[_row_spec(H), _row_spec(H)],
        out_shape=[jax.ShapeDtypeStruct((N, H), jnp.float32)] * 2,
    )(x, w_cat)

    sum1, deg2 = edge_sum_deg(xn.reshape(2 * N, 128), gidx2, row2)
    deg0 = deg2[:n_pad].reshape(n_pad, 1)
    deg1 = deg2[n_pad:].reshape(n_pad, 1)

    x1, y2 = pl.pallas_call(
        _k2_body,
        grid=grid,
        in_specs=[_row_spec(H), _row_spec(H), _row_spec(128), _row_spec(128),
                  _row_spec(1), _row_spec(1), _full_spec((H, H)),
                  _full_spec((H, H)), _full_spec((1, H)), _full_spec((1, H))],
        out_specs=[_row_spec(H), _row_spec(H)],
        out_shape=[jax.ShapeDtypeStruct((N, H), jnp.float32)] * 2,
    )(x_ego, xn, sum1[0], sum1[1], deg0, deg1, W_hop1, W_hop2, s1, t1)

    sum2 = edge_sum_nodeg(y2.reshape(2 * N, 128), gidx2, row2)
    if isinstance(sum2, (list, tuple)):
        sum2 = sum2[0]

    out = pl.pallas_call(
        functools.partial(_k3_body, n_classes=C),
        grid=grid,
        in_specs=[_row_spec(H), _row_spec(H), _row_spec(128), _row_spec(128),
                  _row_spec(1), _row_spec(1), _full_spec((H, 128)),
                  _full_spec((H, 128)), _full_spec((1, 128)),
                  _full_spec((1, H)), _full_spec((1, H))],
        out_specs=_row_spec(128),
        out_shape=jax.ShapeDtypeStruct((N, 128), jnp.float32),
    )(x1, y2, sum2[0], sum2[1], deg0, deg1, wt, wb, bpad, s2, t2)

    return out[:, :C]
